# baseline (speedup 1.0000x reference)
import functools

import jax
import jax.numpy as jnp
from jax import lax
from jax.experimental import pallas as pl
from jax.experimental.pallas import tpu as pltpu
from jax.experimental.pallas import tpu_sc as plsc

V = 1_000_000
B = 128
VL = 1024
CH = 16384
NBLK = V // CH
TAIL = V - NBLK * CH
SUB = 8

_mesh = plsc.VectorSubcoreMesh(
    core_axis_name="c", subcore_axis_name="s", num_cores=1, num_subcores=16
)


@functools.partial(
    pl.kernel,
    out_type=jax.ShapeDtypeStruct((B,), jnp.float32),
    mesh=_mesh,
    scratch_types=[
        pltpu.VMEM((B,), jnp.int32),
        pltpu.VMEM((B,), jnp.float32),
        pltpu.SemaphoreType.DMA,
    ],
)
def _sc_gather(logits_hbm, value_hbm, out_g, idx_v, g_v, sem):
    sid = lax.axis_index("s")
    cid = lax.axis_index("c")

    @pl.when((sid == 0) & (cid == 0))
    def _():
        pltpu.sync_copy(value_hbm, idx_v)
        pltpu.async_copy(logits_hbm.at[idx_v], g_v, sem).wait()
        pltpu.sync_copy(g_v, out_g)


def _norm_body(x_ref, tail_ref, o_ref, stash_ref, macc_ref):
    i = pl.program_id(0)

    m = x_ref[pl.ds(0, VL)]
    for k in range(1, CH // VL):
        m = jnp.maximum(m, x_ref[pl.ds(k * VL, VL)])

    @pl.when(i == 0)
    def _():
        macc_ref[...] = m

    @pl.when(i > 0)
    def _():
        macc_ref[...] = jnp.maximum(macc_ref[...], m)

    stash_ref[pl.ds(i * CH, CH)] = x_ref[...]

    @pl.when(i == NBLK - 1)
    def _():
        tail = tail_ref[...]
        gmax = jnp.maximum(jnp.max(macc_ref[...]), jnp.max(tail))

        def body(j, accs):
            base = j * (SUB * VL)
            return tuple(
                accs[k] + jnp.exp(stash_ref[pl.ds(base + k * VL, VL)] - gmax)
                for k in range(SUB)
            )

        zero = jnp.zeros((VL,), jnp.float32)
        accs = lax.fori_loop(0, (NBLK * CH) // (SUB * VL), body, (zero,) * SUB)
        total = jnp.sum(functools.reduce(jnp.add, accs))
        total = total + jnp.sum(jnp.exp(tail - gmax))
        o_ref[...] = jnp.full((B,), gmax + jnp.log(total), jnp.float32)


def _tc_norm(x1d, tail):
    return pl.pallas_call(
        _norm_body,
        grid=(NBLK,),
        in_specs=[
            pl.BlockSpec((CH,), lambda i: (i,)),
            pl.BlockSpec((TAIL,), lambda i: (0,)),
        ],
        out_specs=pl.BlockSpec((B,), lambda i: (0,)),
        out_shape=jax.ShapeDtypeStruct((B,), jnp.float32),
        scratch_shapes=[
            pltpu.VMEM((NBLK * CH,), jnp.float32),
            pltpu.VMEM((VL,), jnp.float32),
        ],
    )(x1d, tail)


def kernel(logits, value):
    g = _sc_gather(logits, value)
    norm = _tc_norm(logits, lax.slice(logits, (NBLK * CH,), (V,)))
    return g - norm


# speedup vs baseline: 1.6865x; 1.6865x over previous
import functools

import jax
import jax.numpy as jnp
from jax import lax
from jax.experimental import pallas as pl
from jax.experimental.pallas import tpu as pltpu
from jax.experimental.pallas import tpu_sc as plsc

V = 1_000_000
B = 128
VL = 1024
CH = 65536
NBLK = V // CH
TAIL = V - NBLK * CH
FAN = 8

_mesh = plsc.VectorSubcoreMesh(
    core_axis_name="c", subcore_axis_name="s", num_cores=1, num_subcores=16
)


@functools.partial(
    pl.kernel,
    out_type=jax.ShapeDtypeStruct((B,), jnp.float32),
    mesh=_mesh,
    scratch_types=[
        pltpu.VMEM((B,), jnp.int32),
        pltpu.VMEM((B,), jnp.float32),
        pltpu.SemaphoreType.DMA,
    ],
)
def _sc_gather(logits_hbm, value_hbm, out_g, idx_v, g_v, sem):
    sid = lax.axis_index("s")
    cid = lax.axis_index("c")

    @pl.when((sid == 0) & (cid == 0))
    def _():
        pltpu.sync_copy(value_hbm, idx_v)
        pltpu.async_copy(logits_hbm.at[idx_v], g_v, sem).wait()
        pltpu.sync_copy(g_v, out_g)


def _norm_body(x_ref, tail_ref, o_ref, stash_ref, macc_ref):
    i = pl.program_id(0)

    accs = [x_ref[pl.ds(k * VL, VL)] for k in range(FAN)]
    for k in range(FAN, CH // VL):
        accs[k % FAN] = jnp.maximum(accs[k % FAN], x_ref[pl.ds(k * VL, VL)])
    m = functools.reduce(jnp.maximum, accs)

    @pl.when(i == 0)
    def _():
        macc_ref[...] = m

    @pl.when(i > 0)
    def _():
        macc_ref[...] = jnp.maximum(macc_ref[...], m)

    stash_ref[pl.ds(i * CH, CH)] = x_ref[...]

    @pl.when(i == NBLK - 1)
    def _():
        tail = tail_ref[...]
        gmax = jnp.maximum(jnp.max(macc_ref[...]), jnp.max(tail))

        def body(j, accs):
            base = j * (FAN * VL)
            return tuple(
                accs[k] + jnp.exp(stash_ref[pl.ds(base + k * VL, VL)] - gmax)
                for k in range(FAN)
            )

        zero = jnp.zeros((VL,), jnp.float32)
        saccs = lax.fori_loop(0, (NBLK * CH) // (FAN * VL), body, (zero,) * FAN)
        total = jnp.sum(functools.reduce(jnp.add, saccs))
        total = total + jnp.sum(jnp.exp(tail - gmax))
        o_ref[...] = jnp.full((B,), gmax + jnp.log(total), jnp.float32)


def _tc_norm(x1d, tail):
    return pl.pallas_call(
        _norm_body,
        grid=(NBLK,),
        in_specs=[
            pl.BlockSpec((CH,), lambda i: (i,)),
            pl.BlockSpec((TAIL,), lambda i: (0,)),
        ],
        out_specs=pl.BlockSpec((B,), lambda i: (0,)),
        out_shape=jax.ShapeDtypeStruct((B,), jnp.float32),
        scratch_shapes=[
            pltpu.VMEM((NBLK * CH,), jnp.float32),
            pltpu.VMEM((VL,), jnp.float32),
        ],
    )(x1d, tail)


def kernel(logits, value):
    g = _sc_gather(logits, value)
    norm = _tc_norm(logits, lax.slice(logits, (NBLK * CH,), (V,)))
    return g - norm


# speedup vs baseline: 2.1823x; 1.2940x over previous
import functools

import jax
import jax.numpy as jnp
from jax import lax
from jax.experimental import pallas as pl
from jax.experimental.pallas import tpu as pltpu
from jax.experimental.pallas import tpu_sc as plsc

V = 1_000_000
B = 128
VL = 1024
CH = 65536
NBLK = V // CH
TAIL = V - NBLK * CH
FAN = 8

_mesh = plsc.VectorSubcoreMesh(
    core_axis_name="c", subcore_axis_name="s", num_cores=1, num_subcores=16
)


@functools.partial(
    pl.kernel,
    out_type=jax.ShapeDtypeStruct((B,), jnp.float32),
    mesh=_mesh,
    scratch_types=[
        pltpu.VMEM((B,), jnp.int32),
        pltpu.VMEM((B,), jnp.float32),
        pltpu.SemaphoreType.DMA,
    ],
)
def _sc_gather(logits_hbm, value_hbm, out_g, idx_v, g_v, sem):
    sid = lax.axis_index("s")
    cid = lax.axis_index("c")

    @pl.when((sid == 0) & (cid == 0))
    def _():
        pltpu.sync_copy(value_hbm, idx_v)
        pltpu.async_copy(logits_hbm.at[idx_v], g_v, sem).wait()
        pltpu.sync_copy(g_v, out_g)


NG = 122


def _norm_body(x_hbm, o_ref, stash_ref, sem):
    pltpu.make_async_copy(x_hbm, stash_ref, sem).start()
    pltpu.make_async_copy(x_hbm, stash_ref, sem).wait()

    def max_body(j, accs):
        base = j * (FAN * VL)
        return tuple(
            jnp.maximum(accs[k], stash_ref[pl.ds(base + k * VL, VL)])
            for k in range(FAN)
        )

    minf = jnp.full((VL,), -jnp.inf, jnp.float32)
    maccs = lax.fori_loop(0, NG, max_body, (minf,) * FAN)
    tail = stash_ref[pl.ds(NG * FAN * VL, V - NG * FAN * VL)]
    gmax = jnp.maximum(jnp.max(functools.reduce(jnp.maximum, maccs)),
                       jnp.max(tail))

    def sum_body(j, accs):
        base = j * (FAN * VL)
        return tuple(
            accs[k] + jnp.exp(stash_ref[pl.ds(base + k * VL, VL)] - gmax)
            for k in range(FAN)
        )

    zero = jnp.zeros((VL,), jnp.float32)
    saccs = lax.fori_loop(0, NG, sum_body, (zero,) * FAN)
    total = jnp.sum(functools.reduce(jnp.add, saccs))
    total = total + jnp.sum(jnp.exp(tail - gmax))
    o_ref[...] = jnp.full((B,), gmax + jnp.log(total), jnp.float32)


def _tc_norm(x1d):
    return pl.pallas_call(
        _norm_body,
        in_specs=[pl.BlockSpec(memory_space=pl.ANY)],
        out_shape=jax.ShapeDtypeStruct((B,), jnp.float32),
        scratch_shapes=[
            pltpu.VMEM((V,), jnp.float32),
            pltpu.SemaphoreType.DMA,
        ],
    )(x1d)


def kernel(logits, value):
    g = _sc_gather(logits, value)
    norm = _tc_norm(logits)
    return g - norm
